# baseline (speedup 1.0000x reference)
import functools

import jax
import jax.numpy as jnp
from jax import lax
from jax.experimental import pallas as pl
from jax.experimental.pallas import tpu as pltpu
from jax.experimental.pallas import tpu_sc as plsc

B = 16384
F = 32
NC = 2
NS = 16
NW = NC * NS
BPW = B // NW
CHUNK = 128
NCHUNK = BPW // CHUNK
NBUF = 2
L = 16


def _sc_body(users_hbm, movies_hbm, uembt_hbm, membt_hbm, ubias_hbm,
             mbias_hbm, out_hbm, uidx_v, midx_v, ubuf_v, mbuf_v,
             ubias_v, mbias_v, dot_v, gsem0, gsem1, bsem):
    gsems = (gsem0, gsem1)
    wid = lax.axis_index("s") * NC + lax.axis_index("c")
    base = wid * BPW

    for j in range(NCHUNK):
        off = base + j * CHUNK
        pltpu.sync_copy(users_hbm.at[pl.ds(off, CHUNK)], uidx_v.at[j])
        pltpu.sync_copy(movies_hbm.at[pl.ds(off, CHUNK)], midx_v.at[j])

    bias_copies = []
    for j in range(NCHUNK):
        dst = pl.ds(j * CHUNK, CHUNK)
        bias_copies.append(pltpu.async_copy(ubias_hbm.at[uidx_v.at[j]],
                                            ubias_v.at[dst], bsem))
        bias_copies.append(pltpu.async_copy(mbias_hbm.at[midx_v.at[j]],
                                            mbias_v.at[dst], bsem))

    def start_chunk(j):
        b = j % NBUF
        cs = []
        for f in range(F):
            cs.append(pltpu.async_copy(
                uembt_hbm.at[f].at[uidx_v.at[j]], ubuf_v.at[b, f], gsems[b]))
            cs.append(pltpu.async_copy(
                membt_hbm.at[f].at[midx_v.at[j]], mbuf_v.at[b, f], gsems[b]))
        return cs

    def compute_chunk(j):
        b = j % NBUF
        for g in range(CHUNK // L):
            s = pl.ds(j * CHUNK + g * L, L)
            gs = pl.ds(g * L, L)
            acc = ubias_v[s] + mbias_v[s]
            for f in range(F):
                acc += ubuf_v[b, f, gs] * mbuf_v[b, f, gs]
            dot_v[s] = 1.0 / (1.0 + jnp.exp(-acc))

    inflight = [start_chunk(0)]
    for c in bias_copies:
        c.wait()
    for j in range(NCHUNK):
        if j + 1 < NCHUNK:
            inflight.append(start_chunk(j + 1))
        for c in inflight[j]:
            c.wait()
        compute_chunk(j)

    pltpu.sync_copy(dot_v, out_hbm.at[pl.ds(base, BPW)])


@jax.jit
def _mf_sc(users, movies, uembt, membt, ubias1d, mbias1d):
    mesh = plsc.VectorSubcoreMesh(core_axis_name="c", subcore_axis_name="s")
    return pl.kernel(
        _sc_body,
        out_type=jax.ShapeDtypeStruct((B,), jnp.float32),
        mesh=mesh,
        compiler_params=pltpu.CompilerParams(needs_layout_passes=False,
                                             use_tc_tiling_on_sc=False),
        scratch_types=[
            pltpu.VMEM((NCHUNK, CHUNK), jnp.int32),
            pltpu.VMEM((NCHUNK, CHUNK), jnp.int32),
            pltpu.VMEM((NBUF, F, CHUNK), jnp.float32),
            pltpu.VMEM((NBUF, F, CHUNK), jnp.float32),
            pltpu.VMEM((BPW,), jnp.float32),
            pltpu.VMEM((BPW,), jnp.float32),
            pltpu.VMEM((BPW,), jnp.float32),
            pltpu.SemaphoreType.DMA,
            pltpu.SemaphoreType.DMA,
            pltpu.SemaphoreType.DMA,
        ],
    )(users, movies, uembt, membt, ubias1d, mbias1d)


def kernel(users, movies, user_embedding, movie_embedding, user_bias,
           movie_bias):
    return _mf_sc(users, movies, user_embedding.T, movie_embedding.T,
                  user_bias.reshape(-1), movie_bias.reshape(-1))


# speedup vs baseline: 5.6828x; 5.6828x over previous
import functools

import jax
import jax.numpy as jnp
from jax import lax
from jax.experimental import pallas as pl
from jax.experimental.pallas import tpu as pltpu
from jax.experimental.pallas import tpu_sc as plsc

B = 16384
F = 32
LINE = 128
RPL = LINE // F
NC = 2
NS = 16
NW = NC * NS
BPW = B // NW
CHUNK = 128
NCHUNK = BPW // CHUNK
NBUF = 2
L = 16


def _sc_body(users_hbm, movies_hbm, uemb_hbm, memb_hbm, ubias_hbm, mbias_hbm,
             out_hbm, uidx_v, midx_v, uline_v, mline_v, ubuf_v, mbuf_v,
             ubias_v, mbias_v, dot_v, gsem0, gsem1, bsem):
    gsems = (gsem0, gsem1)
    wid = lax.axis_index("s") * NC + lax.axis_index("c")
    base = wid * BPW
    lane = lax.iota(jnp.int32, L)

    for j in range(NCHUNK):
        off = base + j * CHUNK
        pltpu.sync_copy(users_hbm.at[pl.ds(off, CHUNK)], uidx_v.at[j])
        pltpu.sync_copy(movies_hbm.at[pl.ds(off, CHUNK)], midx_v.at[j])

    bias_copies = []
    for j in range(NCHUNK):
        dst = pl.ds(j * CHUNK, CHUNK)
        bias_copies.append(pltpu.async_copy(ubias_hbm.at[uidx_v.at[j]],
                                            ubias_v.at[dst], bsem))
        bias_copies.append(pltpu.async_copy(mbias_hbm.at[midx_v.at[j]],
                                            mbias_v.at[dst], bsem))

    for j in range(NCHUNK):
        for g in range(CHUNK // L):
            s = pl.ds(g * L, L)
            uline_v[j, s] = lax.shift_right_logical(uidx_v[j, s], 2)
            mline_v[j, s] = lax.shift_right_logical(midx_v[j, s], 2)

    def start_chunk(j):
        b = j % NBUF
        return (pltpu.async_copy(uemb_hbm.at[uline_v.at[j]], ubuf_v.at[b],
                                 gsems[b]),
                pltpu.async_copy(memb_hbm.at[mline_v.at[j]], mbuf_v.at[b],
                                 gsems[b]))

    def compute_chunk(j):
        b = j % NBUF
        for g in range(CHUNK // L):
            s = pl.ds(g * L, L)
            uidx = uidx_v[j, s]
            midx = midx_v[j, s]
            ucol = (uidx & (RPL - 1)) * F
            mcol = (midx & (RPL - 1)) * F
            acc = ubias_v[pl.ds(j * CHUNK + g * L, L)] + \
                mbias_v[pl.ds(j * CHUNK + g * L, L)]
            rows = g * L + lane
            for f in range(F):
                acc += (plsc.load_gather(ubuf_v.at[b], [rows, ucol + f]) *
                        plsc.load_gather(mbuf_v.at[b], [rows, mcol + f]))
            dot_v[pl.ds(j * CHUNK + g * L, L)] = 1.0 / (1.0 + jnp.exp(-acc))

    inflight = [start_chunk(0)]
    for c in bias_copies:
        c.wait()
    for j in range(NCHUNK):
        if j + 1 < NCHUNK:
            inflight.append(start_chunk(j + 1))
        for c in inflight[j]:
            c.wait()
        compute_chunk(j)

    pltpu.sync_copy(dot_v, out_hbm.at[pl.ds(base, BPW)])


@jax.jit
def _mf_sc(users, movies, uemb, memb, ubias1d, mbias1d):
    mesh = plsc.VectorSubcoreMesh(core_axis_name="c", subcore_axis_name="s")
    return pl.kernel(
        _sc_body,
        out_type=jax.ShapeDtypeStruct((B,), jnp.float32),
        mesh=mesh,
        compiler_params=pltpu.CompilerParams(needs_layout_passes=False),
        scratch_types=[
            pltpu.VMEM((NCHUNK, CHUNK), jnp.int32),
            pltpu.VMEM((NCHUNK, CHUNK), jnp.int32),
            pltpu.VMEM((NCHUNK, CHUNK), jnp.int32),
            pltpu.VMEM((NCHUNK, CHUNK), jnp.int32),
            pltpu.VMEM((NBUF, CHUNK, LINE), jnp.float32),
            pltpu.VMEM((NBUF, CHUNK, LINE), jnp.float32),
            pltpu.VMEM((BPW,), jnp.float32),
            pltpu.VMEM((BPW,), jnp.float32),
            pltpu.VMEM((BPW,), jnp.float32),
            pltpu.SemaphoreType.DMA,
            pltpu.SemaphoreType.DMA,
            pltpu.SemaphoreType.DMA,
        ],
    )(users, movies, uemb, memb, ubias1d, mbias1d)


def kernel(users, movies, user_embedding, movie_embedding, user_bias,
           movie_bias):
    return _mf_sc(users, movies,
                  user_embedding.reshape(-1, LINE),
                  movie_embedding.reshape(-1, LINE),
                  user_bias.reshape(-1), movie_bias.reshape(-1))


# speedup vs baseline: 12.8568x; 2.2624x over previous
import jax
import jax.numpy as jnp
from jax import lax
from jax.experimental import pallas as pl
from jax.experimental.pallas import tpu as pltpu
from jax.experimental.pallas import tpu_sc as plsc

B = 16384
F = 32
V = 1000000
NC = 2
NS = 16
NW = NC * NS
BPW = B // NW
L = 16
NB = 2


def _body(users_hbm, movies_hbm, uembt_hbm, membt_hbm, ubias_hbm,
          mbias_hbm, out_hbm,
          uidx_s, midx_s, ublk_v, mblk_v, ubb_v, mbb_v, res_v,
          usem, msem, ubsem, mbsem):
    wid = lax.axis_index("s") * NC + lax.axis_index("c")
    base = wid * BPW

    pltpu.sync_copy(users_hbm.at[pl.ds(base, BPW)], uidx_s.at[pl.ds(0, BPW)])
    pltpu.sync_copy(movies_hbm.at[pl.ds(base, BPW)], midx_s.at[pl.ds(0, BPW)])

    def fire(i, b):
        u = uidx_s[pl.ds(i, L)][0]
        m = midx_s[pl.ds(i, L)][0]
        ublo = pl.multiple_of((u // 128) * 128, 128)
        mblo = pl.multiple_of((m // 128) * 128, 128)
        pltpu.async_copy(uembt_hbm.at[:, pl.ds(ublo, 128)], ublk_v.at[b],
                         usem)
        pltpu.async_copy(membt_hbm.at[:, pl.ds(mblo, 128)], mblk_v.at[b],
                         msem)
        pltpu.async_copy(ubias_hbm.at[pl.ds(ublo, 128)], ubb_v.at[b], ubsem)
        pltpu.async_copy(mbias_hbm.at[pl.ds(mblo, 128)], mbb_v.at[b], mbsem)

    for b in range(NB):
        fire(b, b)

    r0 = lax.iota(jnp.int32, L)
    r1 = r0 + L
    lane0 = r0 == 0

    def step(i2, carry):
        for b in range(NB):
            i = i2 * NB + b
            pltpu.make_async_copy(uembt_hbm.at[:, pl.ds(0, 128)],
                                  ublk_v.at[b], usem).wait()
            pltpu.make_async_copy(membt_hbm.at[:, pl.ds(0, 128)],
                                  mblk_v.at[b], msem).wait()
            pltpu.make_async_copy(ubias_hbm.at[pl.ds(0, 128)],
                                  ubb_v.at[b], ubsem).wait()
            pltpu.make_async_copy(mbias_hbm.at[pl.ds(0, 128)],
                                  mbb_v.at[b], mbsem).wait()

            u = uidx_s[pl.ds(i, L)][0]
            m = midx_s[pl.ds(i, L)][0]
            uc = jnp.full((L,), u % 128, jnp.int32)
            mc = jnp.full((L,), m % 128, jnp.int32)
            gu0 = plsc.load_gather(ublk_v.at[b], [r0, uc])
            gu1 = plsc.load_gather(ublk_v.at[b], [r1, uc])
            gm0 = plsc.load_gather(mblk_v.at[b], [r0, mc])
            gm1 = plsc.load_gather(mblk_v.at[b], [r1, mc])
            bu = plsc.load_gather(ubb_v.at[b], [uc])
            bm = plsc.load_gather(mbb_v.at[b], [mc])
            x = jnp.sum(gu0 * gm0 + gu1 * gm1 + (bu + bm) * (1.0 / L),
                        axis=0)
            plsc.store_scatter(res_v, [jnp.full((L,), i, jnp.int32)],
                               jnp.full((L,), x, jnp.float32), mask=lane0)

            @pl.when(i + NB < BPW)
            def _():
                fire(i + NB, b)

        return carry

    lax.fori_loop(0, BPW // NB, step, 0)

    pltpu.sync_copy(res_v, out_hbm.at[pl.ds(base, BPW)])


def _k2_body(x_ref, out_ref):
    out_ref[...] = jax.nn.sigmoid(x_ref[...])


@jax.jit
def _mf(users, movies, uembt, membt, ubias1d, mbias1d):
    mesh = plsc.VectorSubcoreMesh(core_axis_name="c", subcore_axis_name="s")
    scores = pl.kernel(
        _body,
        out_type=jax.ShapeDtypeStruct((B,), jnp.float32),
        mesh=mesh,
        compiler_params=pltpu.CompilerParams(needs_layout_passes=False),
        scratch_types=[
            pltpu.VMEM((BPW + L,), jnp.int32),
            pltpu.VMEM((BPW + L,), jnp.int32),
            pltpu.VMEM((NB, F, 128), jnp.float32),
            pltpu.VMEM((NB, F, 128), jnp.float32),
            pltpu.VMEM((NB, 128), jnp.float32),
            pltpu.VMEM((NB, 128), jnp.float32),
            pltpu.VMEM((BPW,), jnp.float32),
            pltpu.SemaphoreType.DMA,
            pltpu.SemaphoreType.DMA,
            pltpu.SemaphoreType.DMA,
            pltpu.SemaphoreType.DMA,
        ],
    )(users, movies, uembt, membt, ubias1d, mbias1d)

    out2 = pl.pallas_call(
        _k2_body,
        out_shape=jax.ShapeDtypeStruct((1, B), jnp.float32),
    )(scores.reshape(1, B))
    return out2.reshape(B)


def kernel(users, movies, user_embedding, movie_embedding, user_bias,
           movie_bias):
    return _mf(users, movies, user_embedding.T, movie_embedding.T,
               user_bias.reshape(-1), movie_bias.reshape(-1))


# speedup vs baseline: 17.8441x; 1.3879x over previous
import jax
import jax.numpy as jnp
from jax import lax
from jax.experimental import pallas as pl
from jax.experimental.pallas import tpu as pltpu
from jax.experimental.pallas import tpu_sc as plsc

B = 16384
F = 32
V = 1000000
NC = 2
NS = 16
NW = NC * NS
BPW = B // NW
L = 16
NB = 8


def _body(users_hbm, movies_hbm, uembt_hbm, membt_hbm, ubias_hbm,
          mbias_hbm, out_hbm,
          uidx_s, midx_s, ublk_v, mblk_v, ubb_v, mbb_v, res_v,
          usem, msem, ubsem, mbsem):
    wid = lax.axis_index("s") * NC + lax.axis_index("c")
    base = wid * BPW

    pltpu.sync_copy(users_hbm.at[pl.ds(base, BPW)], uidx_s.at[pl.ds(0, BPW)])
    pltpu.sync_copy(movies_hbm.at[pl.ds(base, BPW)], midx_s.at[pl.ds(0, BPW)])

    def fire(i, b):
        u = uidx_s[pl.ds(i, L)][0]
        m = midx_s[pl.ds(i, L)][0]
        ublo = pl.multiple_of((u // 128) * 128, 128)
        mblo = pl.multiple_of((m // 128) * 128, 128)
        pltpu.async_copy(uembt_hbm.at[:, pl.ds(ublo, 128)], ublk_v.at[b],
                         usem)
        pltpu.async_copy(membt_hbm.at[:, pl.ds(mblo, 128)], mblk_v.at[b],
                         msem)
        pltpu.async_copy(ubias_hbm.at[pl.ds(ublo, 128)], ubb_v.at[b], ubsem)
        pltpu.async_copy(mbias_hbm.at[pl.ds(mblo, 128)], mbb_v.at[b], mbsem)

    for b in range(NB):
        fire(b, b)

    r0 = lax.iota(jnp.int32, L)
    r1 = r0 + L
    lane0 = r0 == 0

    def step(i2, carry):
        for b in range(NB):
            i = i2 * NB + b
            pltpu.make_async_copy(uembt_hbm.at[:, pl.ds(0, 128)],
                                  ublk_v.at[b], usem).wait()
            pltpu.make_async_copy(membt_hbm.at[:, pl.ds(0, 128)],
                                  mblk_v.at[b], msem).wait()
            pltpu.make_async_copy(ubias_hbm.at[pl.ds(0, 128)],
                                  ubb_v.at[b], ubsem).wait()
            pltpu.make_async_copy(mbias_hbm.at[pl.ds(0, 128)],
                                  mbb_v.at[b], mbsem).wait()

            u = uidx_s[pl.ds(i, L)][0]
            m = midx_s[pl.ds(i, L)][0]
            uc = jnp.full((L,), u % 128, jnp.int32)
            mc = jnp.full((L,), m % 128, jnp.int32)
            gu0 = plsc.load_gather(ublk_v.at[b], [r0, uc])
            gu1 = plsc.load_gather(ublk_v.at[b], [r1, uc])
            gm0 = plsc.load_gather(mblk_v.at[b], [r0, mc])
            gm1 = plsc.load_gather(mblk_v.at[b], [r1, mc])
            bu = plsc.load_gather(ubb_v.at[b], [uc])
            bm = plsc.load_gather(mbb_v.at[b], [mc])
            x = jnp.sum(gu0 * gm0 + gu1 * gm1 + (bu + bm) * (1.0 / L),
                        axis=0)
            xs = jnp.full((L,), x, jnp.float32)
            sig = 1.0 / (1.0 + jnp.exp(-xs))
            plsc.store_scatter(res_v, [jnp.full((L,), i, jnp.int32)],
                               sig, mask=lane0)

            @pl.when(i + NB < BPW)
            def _():
                fire(i + NB, b)

        return carry

    lax.fori_loop(0, BPW // NB, step, 0)

    pltpu.sync_copy(res_v, out_hbm.at[pl.ds(base, BPW)])


@jax.jit
def _mf(users, movies, uembt, membt, ubias1d, mbias1d):
    mesh = plsc.VectorSubcoreMesh(core_axis_name="c", subcore_axis_name="s")
    scores = pl.kernel(
        _body,
        out_type=jax.ShapeDtypeStruct((B,), jnp.float32),
        mesh=mesh,
        compiler_params=pltpu.CompilerParams(needs_layout_passes=False),
        scratch_types=[
            pltpu.VMEM((BPW + L,), jnp.int32),
            pltpu.VMEM((BPW + L,), jnp.int32),
            pltpu.VMEM((NB, F, 128), jnp.float32),
            pltpu.VMEM((NB, F, 128), jnp.float32),
            pltpu.VMEM((NB, 128), jnp.float32),
            pltpu.VMEM((NB, 128), jnp.float32),
            pltpu.VMEM((BPW,), jnp.float32),
            pltpu.SemaphoreType.DMA,
            pltpu.SemaphoreType.DMA,
            pltpu.SemaphoreType.DMA,
            pltpu.SemaphoreType.DMA,
        ],
    )(users, movies, uembt, membt, ubias1d, mbias1d)

    return scores


def kernel(users, movies, user_embedding, movie_embedding, user_bias,
           movie_bias):
    return _mf(users, movies, user_embedding.T, movie_embedding.T,
               user_bias.reshape(-1), movie_bias.reshape(-1))


# speedup vs baseline: 17.8756x; 1.0018x over previous
import jax
import jax.numpy as jnp
from jax import lax
from jax.experimental import pallas as pl
from jax.experimental.pallas import tpu as pltpu
from jax.experimental.pallas import tpu_sc as plsc

B = 16384
F = 32
V = 1000000
NC = 2
NS = 16
NW = NC * NS
BPW = B // NW
L = 16
NB = 8


def _body(users_hbm, movies_hbm, uembt_hbm, membt_hbm, ubias_hbm,
          mbias_hbm, out_hbm,
          uidx_s, midx_s, ublk_v, mblk_v, ubb_v, mbb_v, res_v,
          usem, msem, ubsem, mbsem):
    wid = lax.axis_index("s") * NC + lax.axis_index("c")
    base = wid * BPW

    pltpu.sync_copy(users_hbm.at[pl.ds(base, BPW)], uidx_s.at[pl.ds(0, BPW)])
    pltpu.sync_copy(movies_hbm.at[pl.ds(base, BPW)], midx_s.at[pl.ds(0, BPW)])

    def fire(u, m, b):
        ublo = pl.multiple_of((u // 128) * 128, 128)
        mblo = pl.multiple_of((m // 128) * 128, 128)
        pltpu.async_copy(uembt_hbm.at[:, pl.ds(ublo, 128)], ublk_v.at[b],
                         usem)
        pltpu.async_copy(membt_hbm.at[:, pl.ds(mblo, 128)], mblk_v.at[b],
                         msem)
        pltpu.async_copy(ubias_hbm.at[pl.ds(ublo, 128)], ubb_v.at[b], ubsem)
        pltpu.async_copy(mbias_hbm.at[pl.ds(mblo, 128)], mbb_v.at[b], mbsem)

    uvec0 = uidx_s[pl.ds(0, L)]
    mvec0 = midx_s[pl.ds(0, L)]
    for b in range(NB):
        fire(uvec0[b], mvec0[b], b)

    r0 = lax.iota(jnp.int32, L)
    r1 = r0 + L

    def step(g, carry):
        i0 = g * L
        uvec = uidx_s[pl.ds(i0, L)]
        mvec = midx_s[pl.ds(i0, L)]
        uvecn = uidx_s[pl.ds(i0 + L, L)]
        mvecn = midx_s[pl.ds(i0 + L, L)]
        resvec = jnp.zeros((L,), jnp.float32)

        for j in range(L):
            i = i0 + j
            b = j % NB
            pltpu.make_async_copy(uembt_hbm.at[:, pl.ds(0, 128)],
                                  ublk_v.at[b], usem).wait()
            pltpu.make_async_copy(membt_hbm.at[:, pl.ds(0, 128)],
                                  mblk_v.at[b], msem).wait()
            pltpu.make_async_copy(ubias_hbm.at[pl.ds(0, 128)],
                                  ubb_v.at[b], ubsem).wait()
            pltpu.make_async_copy(mbias_hbm.at[pl.ds(0, 128)],
                                  mbb_v.at[b], mbsem).wait()

            uc = jnp.full((L,), uvec[j] % 128, jnp.int32)
            mc = jnp.full((L,), mvec[j] % 128, jnp.int32)
            gu0 = plsc.load_gather(ublk_v.at[b], [r0, uc])
            gu1 = plsc.load_gather(ublk_v.at[b], [r1, uc])
            gm0 = plsc.load_gather(mblk_v.at[b], [r0, mc])
            gm1 = plsc.load_gather(mblk_v.at[b], [r1, mc])
            bu = plsc.load_gather(ubb_v.at[b], [uc])
            bm = plsc.load_gather(mbb_v.at[b], [mc])
            x = jnp.sum(gu0 * gm0 + gu1 * gm1 + (bu + bm) * (1.0 / L),
                        axis=0)
            xs = jnp.full((L,), x, jnp.float32)
            sig = 1.0 / (1.0 + jnp.exp(-xs))
            resvec = jnp.where(r0 == j, sig, resvec)

            un = uvec[j + NB] if j + NB < L else uvecn[j + NB - L]
            mn = mvec[j + NB] if j + NB < L else mvecn[j + NB - L]

            @pl.when(i + NB < BPW)
            def _():
                fire(un, mn, b)

        res_v[pl.ds(i0, L)] = resvec
        return carry

    lax.fori_loop(0, BPW // L, step, 0)

    pltpu.sync_copy(res_v, out_hbm.at[pl.ds(base, BPW)])


@jax.jit
def _mf(users, movies, uembt, membt, ubias1d, mbias1d):
    mesh = plsc.VectorSubcoreMesh(core_axis_name="c", subcore_axis_name="s")
    scores = pl.kernel(
        _body,
        out_type=jax.ShapeDtypeStruct((B,), jnp.float32),
        mesh=mesh,
        compiler_params=pltpu.CompilerParams(needs_layout_passes=False),
        scratch_types=[
            pltpu.VMEM((BPW + 2 * L,), jnp.int32),
            pltpu.VMEM((BPW + 2 * L,), jnp.int32),
            pltpu.VMEM((NB, F, 128), jnp.float32),
            pltpu.VMEM((NB, F, 128), jnp.float32),
            pltpu.VMEM((NB, 128), jnp.float32),
            pltpu.VMEM((NB, 128), jnp.float32),
            pltpu.VMEM((BPW,), jnp.float32),
            pltpu.SemaphoreType.DMA,
            pltpu.SemaphoreType.DMA,
            pltpu.SemaphoreType.DMA,
            pltpu.SemaphoreType.DMA,
        ],
    )(users, movies, uembt, membt, ubias1d, mbias1d)
    return scores


def kernel(users, movies, user_embedding, movie_embedding, user_bias,
           movie_bias):
    return _mf(users, movies, user_embedding.T, movie_embedding.T,
               user_bias.reshape(-1), movie_bias.reshape(-1))
